# baseline (speedup 1.0000x reference)
import numpy as np

import jax
import jax.numpy as jnp
from jax import lax
from jax.experimental import pallas as pl

B = 16384
P = 4
RP = B // P
EPS = 1e-5

_NS = 54 * P
_NF = 33 * P

_SELC = np.zeros((10 * P, _NS + _NF), np.float32)
for _c in range(P):
    _SELC[10 * _c + 8, 54 * _c:54 * _c + 54] = 1.0
    _SELC[10 * _c + 9, _NS + 33 * _c:_NS + 33 * _c + 33] = 1.0
_VC = np.concatenate([np.tile(np.arange(54, dtype=np.float32), P),
                      np.tile(np.arange(33, dtype=np.float32), P)])[None, :]


def _blockdiag(w, n):
    cols = w.shape[1]
    return jnp.concatenate(
        [jnp.pad(w, ((0, 0), (cols * c, cols * (n - 1 - c)))) for c in range(n)],
        axis=0)


def _bn_scale_shift(h, ones_row, g, be, width):
    m = jnp.dot(ones_row, h, preferred_element_type=jnp.float32) * (1.0 / RP)
    q = jnp.dot(ones_row, h * h, preferred_element_type=jnp.float32) * (1.0 / RP)
    mc = sum(m[:, width * c:width * (c + 1)] for c in range(P)) * (1.0 / P)
    qc = sum(q[:, width * c:width * (c + 1)] for c in range(P)) * (1.0 / P)
    var = qc - mc * mc
    scale = g * lax.rsqrt(var + EPS)
    shift = be - mc * scale
    return (jnp.concatenate([scale] * P, axis=1),
            jnp.concatenate([shift] * P, axis=1))


def _body(X_ref, ft_ref, st_ref, W1_ref, b1_ref, g1_ref, be1_ref,
          W2_ref, b2_ref, g2_ref, be2_ref, W3_ref, b3_ref,
          selc_ref, vc_ref, out_ref):
    X = X_ref[...]
    Xp = jnp.concatenate([X[RP * c:RP * (c + 1), :] for c in range(P)],
                         axis=1)

    sv = jnp.dot(Xp, selc_ref[...], preferred_element_type=jnp.float32)
    oh = (sv == vc_ref[...]).astype(jnp.float32)

    W1 = W1_ref[...]
    stW = jnp.dot(st_ref[...], W1[23:38], preferred_element_type=jnp.float32)
    ftW = jnp.dot(ft_ref[...], W1[8:23], preferred_element_type=jnp.float32)
    tw = jnp.concatenate([_blockdiag(stW, P), _blockdiag(ftW, P)], axis=0)
    W1a10 = jnp.concatenate([W1[0:8], jnp.zeros((2, 20), jnp.float32)], axis=0)

    ones_row = jnp.ones((1, RP), jnp.float32)

    h = (jnp.dot(Xp, _blockdiag(W1a10, P), preferred_element_type=jnp.float32)
         + jnp.dot(oh, tw, preferred_element_type=jnp.float32)
         + jnp.concatenate([b1_ref[...]] * P, axis=1))
    h = jnp.maximum(h, 0.0)
    scale, shift = _bn_scale_shift(h, ones_row, g1_ref[...], be1_ref[...], 20)
    h = h * scale + shift

    h = (jnp.dot(h, _blockdiag(W2_ref[...], P), preferred_element_type=jnp.float32)
         + jnp.concatenate([b2_ref[...]] * P, axis=1))
    h = jnp.maximum(h, 0.0)
    scale2, shift2 = _bn_scale_shift(h, ones_row, g2_ref[...], be2_ref[...], 10)
    h = h * scale2 + shift2

    o_t = jax.lax.dot_general(_blockdiag(W3_ref[...], P), h,
                              (((0,), (1,)), ((), ())),
                              preferred_element_type=jnp.float32)
    out_ref[...] = o_t + b3_ref[...]


def kernel(X, family_table, store_table, W1, b1, g1, be1, W2, b2, g2, be2, W3, b3):
    args = (X, family_table, store_table, W1,
            b1.reshape(1, -1), g1.reshape(1, -1), be1.reshape(1, -1),
            W2, b2.reshape(1, -1), g2.reshape(1, -1), be2.reshape(1, -1),
            W3, b3.reshape(1, -1),
            jnp.asarray(_SELC), jnp.asarray(_VC))
    o_t = pl.pallas_call(
        _body,
        out_shape=jax.ShapeDtypeStruct((P, RP), jnp.float32),
    )(*args)
    return o_t.reshape(B, 1)
